# baseline (speedup 1.0000x reference)
import functools

import jax
import jax.numpy as jnp
from jax import lax
from jax.experimental import pallas as pl
from jax.experimental.pallas import tpu as pltpu
from jax.experimental.pallas import tpu_sc as plsc

NW = 32
L = 16


@functools.partial(jax.jit, static_argnums=(2, 3, 4, 5))
def _emb_lookup(idx1, table, S0, S1, D, B):
    mesh = plsc.VectorSubcoreMesh(core_axis_name="c", subcore_axis_name="s")
    rows_per_w = B // NW
    blk = S0 // NW

    @functools.partial(
        pl.kernel,
        mesh=mesh,
        out_type=jax.ShapeDtypeStruct((S1, D, S0), jnp.float32),
        scratch_types=[
            pltpu.VMEM((rows_per_w,), jnp.int32),
            pltpu.VMEM((S1 * blk,), jnp.int32),
            pltpu.VMEM((2, blk, D), jnp.float32),
            pltpu.VMEM((2, D, blk), jnp.float32),
            pltpu.SemaphoreType.DMA,
            pltpu.SemaphoreType.DMA,
            pltpu.SemaphoreType.DMA,
        ],
        compiler_params=pltpu.CompilerParams(needs_layout_passes=False),
    )
    def emb(idx_hbm, table_hbm, out_hbm, idx_v, idx_p, rowbuf, tstage, ssem,
            gsem, wsem):
        wid = lax.axis_index("s") * 2 + lax.axis_index("c")
        base = wid * rows_per_w
        lane0 = wid * blk
        pltpu.async_copy(idx_hbm.at[pl.ds(base, rows_per_w)], idx_v,
                         ssem).wait()

        def perm(s1, carry):
            for m in range(blk // L):
                rows = lax.iota(jnp.int32, L) * S1 + (m * L * S1 + s1)
                vals = plsc.load_gather(idx_v, [rows])
                idx_p[pl.ds(s1 * blk + m * L, L)] = vals
            return carry

        lax.fori_loop(0, S1, perm, 0)

        def issue(u, b):
            for g in range(blk // L):
                vs = idx_p[pl.ds(u * blk + g * L, L)]
                tv = vs >> 3
                rv = vs & 7
                for k in range(L):
                    pltpu.async_copy(table_hbm.at[tv[k], rv[k]],
                                     rowbuf.at[b, g * L + k], gsem)

        def drain():
            for _ in range(blk):
                pltpu.make_async_copy(table_hbm.at[0, 0], rowbuf.at[0, 0],
                                      gsem).wait()

        def transpose(b):
            def col(j, carry):
                for m in range(blk // L):
                    rows = lax.iota(jnp.int32, L) + m * L
                    vals = plsc.load_gather(rowbuf.at[b],
                                            [rows, jnp.full((L,), j,
                                                            jnp.int32)])
                    tstage[b, j, pl.ds(m * L, L)] = vals
                return carry

            lax.fori_loop(0, D, col, 0)

        def write(u, b):
            pltpu.async_copy(tstage.at[b], out_hbm.at[u, :, pl.ds(lane0, blk)],
                             wsem)

        issue(0, 0)

        def pair(h, carry):
            for b in range(2):
                u = h * 2 + b
                drain()

                @pl.when(u + 1 < S1)
                def _():
                    issue(u + 1, (b + 1) % 2)

                @pl.when(h > 0)
                def _():
                    pltpu.make_async_copy(tstage.at[b],
                                          out_hbm.at[0, :, pl.ds(0, blk)],
                                          wsem).wait()

                transpose(b)
                write(u, b)
            return carry

        lax.fori_loop(0, S1 // 2, pair, 0)
        for b in range(2):
            pltpu.make_async_copy(tstage.at[b], out_hbm.at[0, :, pl.ds(0, blk)],
                                  wsem).wait()

    return emb(idx1, table)


def kernel(indices, table):
    S0, S1 = indices.shape
    V, D = table.shape
    B = S0 * S1
    assert S0 % (NW * L) == 0 and D % L == 0 and S1 % 2 == 0 and V % 8 == 0
    idx1 = indices.astype(jnp.int32).reshape(B)
    table3 = table.reshape(V // 8, 8, D)
    out_k = _emb_lookup(idx1, table3, S0, S1, D, B)
    return out_k.transpose(2, 0, 1)


# speedup vs baseline: 1.1529x; 1.1529x over previous
import functools

import jax
import jax.numpy as jnp
from jax import lax
from jax.experimental import pallas as pl
from jax.experimental.pallas import tpu as pltpu
from jax.experimental.pallas import tpu_sc as plsc

NW = 32
L = 16


@functools.partial(jax.jit, static_argnums=(2, 3, 4, 5))
def _emb_lookup(idx1, table3, S0, S1, D, B):
    mesh = plsc.VectorSubcoreMesh(core_axis_name="c", subcore_axis_name="s")
    rows_per_w = B // NW
    n_slabs = rows_per_w // S1
    n_grp = (S1 + L - 1) // L

    @functools.partial(
        pl.kernel,
        mesh=mesh,
        out_type=jax.ShapeDtypeStruct((S0, S1, D), jnp.float32),
        scratch_types=[
            pltpu.VMEM((rows_per_w + L,), jnp.int32),
            pltpu.VMEM((2, S1 * D), jnp.float32),
            pltpu.VMEM((2, S1, D), jnp.float32),
            pltpu.SemaphoreType.DMA,
            pltpu.SemaphoreType.DMA((4,)),
            pltpu.SemaphoreType.DMA,
        ],
    )
    def emb(idx_hbm, table_hbm, out_hbm, idx_v, flat, stage, ssem, gsems,
            wsem):
        wid = lax.axis_index("s") * 2 + lax.axis_index("c")
        base = wid * rows_per_w
        s0_base = wid * n_slabs
        pltpu.async_copy(idx_hbm.at[pl.ds(base, rows_per_w)],
                         idx_v.at[pl.ds(0, rows_per_w)], ssem).wait()

        def pair(jh, carry):
            for b in range(2):
                j = jh * 2 + b

                @pl.when(jh > 0)
                def _():
                    pltpu.make_async_copy(stage.at[b], out_hbm.at[0],
                                          wsem).wait()

                for g in range(n_grp):
                    vs = idx_v[pl.ds(j * S1 + g * L, L)]
                    tv = vs >> 3
                    rv = vs & 7
                    for k in range(min(L, S1 - g * L)):
                        r = g * L + k
                        pltpu.async_copy(table_hbm.at[tv[k], rv[k]],
                                         flat.at[b, pl.ds(r * D, D)],
                                         gsems.at[r % 4])

                for q in range(4):
                    for _ in range((S1 - q + 3) // 4):
                        pltpu.make_async_copy(table_hbm.at[0, 0],
                                              flat.at[0, pl.ds(0, D)],
                                              gsems.at[q]).wait()

                def repack(r, carry):
                    for c in range(D // L):
                        stage[b, r, pl.ds(c * L, L)] = (
                            flat[b, pl.ds(r * D + c * L, L)])
                    return carry

                lax.fori_loop(0, S1, repack, 0)

                pltpu.async_copy(stage.at[b], out_hbm.at[s0_base + j], wsem)
            return carry

        lax.fori_loop(0, n_slabs // 2, pair, 0)
        for b in range(2):
            pltpu.make_async_copy(stage.at[b], out_hbm.at[0], wsem).wait()

    return emb(idx1, table3)


def kernel(indices, table):
    S0, S1 = indices.shape
    V, D = table.shape
    B = S0 * S1
    assert B % (NW * S1) == 0 and V % 8 == 0 and D % L == 0
    assert (B // (NW * S1)) % 2 == 0
    idx1 = indices.astype(jnp.int32).reshape(B)
    table3 = table.reshape(V // 8, 8, D)
    return _emb_lookup(idx1, table3, S0, S1, D, B)
